# baseline (speedup 1.0000x reference)
import functools

import jax
import jax.numpy as jnp
from jax import lax
from jax.experimental import pallas as pl
from jax.experimental.pallas import tpu as pltpu
from jax.experimental.pallas import tpu_sc as plsc

B = 4096
L = 200
EMBED = 64
HIDDEN = 128

NUM_CORES = 2
NUM_SUBCORES = 16
NW = NUM_CORES * NUM_SUBCORES
SENT_PER_W = B // NW
CHUNK0 = 128
CHUNK1 = L - CHUNK0


NBUF = 2
RUNROLL = 8
VOCAB = 1000000
TCC = 2048


def _transpose_body(tt_ref, out_ref):
    t = tt_ref[...]
    tt = jnp.transpose(t)
    t3 = tt.reshape(TCC // 2, 2, EMBED)
    out_ref[...] = jnp.concatenate([t3[:, 0, :], t3[:, 1, :]], axis=1)


_BCAST_DNUMS = lax.GatherDimensionNumbers(
    offset_dims=(), collapsed_slice_dims=(0,), start_index_map=(0,))


def _gather_sums_body(pairs_hbm, x_hbm, sums_hbm, idx_v, idxp_v, rows_v,
                      acc_v, sem0, sem1):
    sems = (sem0, sem1)
    wid = lax.axis_index("s") * NUM_CORES + lax.axis_index("c")
    base = wid * SENT_PER_W
    pltpu.sync_copy(x_hbm.at[pl.ds(base * L, SENT_PER_W * L)],
                    idx_v.at[pl.ds(0, SENT_PER_W * L)])

    def shift_body(j, carry):
        idxp_v[pl.ds(j * 16, 16)] = idx_v[pl.ds(j * 16, 16)] >> 1
        return carry

    lax.fori_loop(0, SENT_PER_W * L // 16, shift_body, 0)

    def fire(s, b):
        off = pl.multiple_of(s * L, 8)
        pltpu.async_copy(
            pairs_hbm.at[idxp_v.at[pl.ds(off, CHUNK0)]],
            rows_v.at[b, pl.ds(0, CHUNK0)], sems[b])
        pltpu.async_copy(
            pairs_hbm.at[idxp_v.at[pl.ds(off + CHUNK0, CHUNK1)]],
            rows_v.at[b, pl.ds(CHUNK0, CHUNK1)], sems[b])

    def wait(b):
        pltpu.make_async_copy(
            pairs_hbm.at[idxp_v.at[pl.ds(0, CHUNK0)]],
            rows_v.at[b, pl.ds(0, CHUNK0)], sems[b]).wait()
        pltpu.make_async_copy(
            pairs_hbm.at[idxp_v.at[pl.ds(0, CHUNK1)]],
            rows_v.at[b, pl.ds(CHUNK0, CHUNK1)], sems[b]).wait()

    def reduce(s, b):
        off = pl.multiple_of(s * L, 8)

        def red(t, accs):
            r0 = t * RUNROLL
            par16 = ((idx_v[pl.ds(off + r0, 16)] & 1)).astype(jnp.float32)
            for dr in range(RUNROLL):
                p = lax.gather(
                    par16, jnp.full((16, 1), dr, jnp.int32),
                    _BCAST_DNUMS, (1,),
                    mode=lax.GatherScatterMode.PROMISE_IN_BOUNDS)
                for c in range(EMBED // 16):
                    h0 = rows_v[b, r0 + dr, pl.ds(c * 16, 16)]
                    h1 = rows_v[b, r0 + dr, pl.ds(EMBED + c * 16, 16)]
                    accs = (accs[:c]
                            + (accs[c] + h0 + (h1 - h0) * p,)
                            + accs[c + 1:])
            return accs

        z = jnp.zeros((16,), jnp.float32)
        accs = lax.fori_loop(0, L // RUNROLL, red, (z, z, z, z))
        for c in range(EMBED // 16):
            acc_v[s, pl.ds(c * 16, 16)] = accs[c]

    for b in range(NBUF):
        fire(b, b)

    def group(g, carry):
        s0 = g * NBUF
        for b in range(NBUF):
            wait(b)
            reduce(s0 + b, b)
            fire(s0 + b + NBUF, b)
        return carry

    lax.fori_loop(0, SENT_PER_W // NBUF - 1, group, 0)
    s0 = SENT_PER_W - NBUF
    for b in range(NBUF):
        wait(b)
        reduce(s0 + b, b)

    pltpu.sync_copy(acc_v, sums_hbm.at[pl.ds(base, SENT_PER_W)])


_gather_sums = functools.partial(
    pl.kernel,
    out_type=jax.ShapeDtypeStruct((B, EMBED), jnp.float32),
    mesh=plsc.VectorSubcoreMesh(core_axis_name="c", subcore_axis_name="s",
                                num_cores=NUM_CORES,
                                num_subcores=NUM_SUBCORES),
    compiler_params=pltpu.CompilerParams(use_tc_tiling_on_sc=False),
    scratch_types=[
        pltpu.VMEM((SENT_PER_W * L + 16,), jnp.int32),
        pltpu.VMEM((SENT_PER_W * L,), jnp.int32),
        pltpu.VMEM((NBUF, L, 2 * EMBED), jnp.float32),
        pltpu.VMEM((SENT_PER_W, EMBED), jnp.float32),
        pltpu.SemaphoreType.DMA,
        pltpu.SemaphoreType.DMA,
    ],
)(_gather_sums_body)


def _mlp_body(x_ref, sums_ref, w1_ref, b1_ref, w2_ref, b2_ref, out_ref):
    cnt = jnp.sum((x_ref[...] != 0).astype(jnp.float32), axis=1, keepdims=True)
    pooled = sums_ref[...] / (cnt + 1e-8)
    h = jnp.maximum(
        jnp.dot(pooled, w1_ref[...], preferred_element_type=jnp.float32)
        + b1_ref[...], 0.0)
    out_ref[...] = (
        jnp.dot(h, w2_ref[...], preferred_element_type=jnp.float32)
        + b2_ref[...])


BLK = 512


def kernel(x, table, W1, b1, W2, b2):
    pairs = pl.pallas_call(
        _transpose_body,
        grid=(VOCAB // TCC,),
        in_specs=[pl.BlockSpec((EMBED, TCC), lambda i: (0, i))],
        out_specs=pl.BlockSpec((TCC // 2, 2 * EMBED), lambda i: (i, 0)),
        out_shape=jax.ShapeDtypeStruct((VOCAB // 2, 2 * EMBED), jnp.float32),
    )(table.T)
    sums = _gather_sums(pairs, x.reshape(-1))
    out = pl.pallas_call(
        _mlp_body,
        grid=(B // BLK,),
        in_specs=[
            pl.BlockSpec((BLK, L), lambda i: (i, 0)),
            pl.BlockSpec((BLK, EMBED), lambda i: (i, 0)),
            pl.BlockSpec((EMBED, HIDDEN), lambda i: (0, 0)),
            pl.BlockSpec((1, HIDDEN), lambda i: (0, 0)),
            pl.BlockSpec((HIDDEN, EMBED), lambda i: (0, 0)),
            pl.BlockSpec((1, EMBED), lambda i: (0, 0)),
        ],
        out_specs=pl.BlockSpec((BLK, EMBED), lambda i: (i, 0)),
        out_shape=jax.ShapeDtypeStruct((B, EMBED), jnp.float32),
    )(x, sums, W1, b1.reshape(1, HIDDEN), W2, b2.reshape(1, EMBED))
    return out


# speedup vs baseline: 1.1144x; 1.1144x over previous
import functools

import jax
import jax.numpy as jnp
from jax import lax
from jax.experimental import pallas as pl
from jax.experimental.pallas import tpu as pltpu
from jax.experimental.pallas import tpu_sc as plsc

B = 4096
L = 200
EMBED = 64
HIDDEN = 128

NUM_CORES = 2
NUM_SUBCORES = 16
NW = NUM_CORES * NUM_SUBCORES
SENT_PER_W = B // NW
CHUNK0 = 128
CHUNK1 = L - CHUNK0


NBUF = 2
RUNROLL = 8
VOCAB = 1000000
TCC = 2048


def _transpose_body(tt_ref, out_ref):
    t = tt_ref[...]
    ez = jnp.concatenate(
        [jnp.eye(EMBED, dtype=jnp.float32),
         jnp.zeros((EMBED, EMBED), jnp.float32)], axis=1)
    out_ref[...] = lax.dot_general(
        t, ez, (((0,), (0,)), ((), ())),
        preferred_element_type=jnp.float32)


def _gather_sums_body(pairs_hbm, x_hbm, sums_hbm, idx_v, rows_v,
                      acc_v, sem0, sem1, sem2, sem3):
    sems = (sem0, sem1, sem2, sem3)
    wid = lax.axis_index("s") * NUM_CORES + lax.axis_index("c")
    base = wid * SENT_PER_W
    pltpu.sync_copy(x_hbm.at[pl.ds(base * L, SENT_PER_W * L)], idx_v)

    def fire(s, b):
        off = pl.multiple_of(s * L, 8)
        pltpu.async_copy(
            pairs_hbm.at[idx_v.at[pl.ds(off, CHUNK0)]],
            rows_v.at[b, pl.ds(0, CHUNK0)], sems[b])
        pltpu.async_copy(
            pairs_hbm.at[idx_v.at[pl.ds(off + CHUNK0, CHUNK1)]],
            rows_v.at[b, pl.ds(CHUNK0, CHUNK1)], sems[b])

    def wait(b):
        pltpu.make_async_copy(
            pairs_hbm.at[idx_v.at[pl.ds(0, CHUNK0)]],
            rows_v.at[b, pl.ds(0, CHUNK0)], sems[b]).wait()
        pltpu.make_async_copy(
            pairs_hbm.at[idx_v.at[pl.ds(0, CHUNK1)]],
            rows_v.at[b, pl.ds(CHUNK0, CHUNK1)], sems[b]).wait()

    def reduce(s, b):
        def red(t, accs):
            r0 = t * RUNROLL
            for dr in range(RUNROLL):
                accs = tuple(accs[c] + rows_v[b, r0 + dr, pl.ds(c * 16, 16)]
                             for c in range(EMBED // 16))
            return accs

        z = jnp.zeros((16,), jnp.float32)
        accs = lax.fori_loop(0, L // RUNROLL, red, (z, z, z, z))
        for c in range(EMBED // 16):
            acc_v[s, pl.ds(c * 16, 16)] = accs[c]

    for b in range(NBUF):
        fire(b, b)

    def group(g, carry):
        s0 = g * NBUF
        for b in range(NBUF):
            wait(b)
            reduce(s0 + b, b)
            fire(s0 + b + NBUF, b)
        return carry

    lax.fori_loop(0, SENT_PER_W // NBUF - 1, group, 0)
    s0 = SENT_PER_W - NBUF
    for b in range(NBUF):
        wait(b)
        reduce(s0 + b, b)

    pltpu.sync_copy(acc_v, sums_hbm.at[pl.ds(base, SENT_PER_W)])


_gather_sums = functools.partial(
    pl.kernel,
    out_type=jax.ShapeDtypeStruct((B, EMBED), jnp.float32),
    mesh=plsc.VectorSubcoreMesh(core_axis_name="c", subcore_axis_name="s",
                                num_cores=NUM_CORES,
                                num_subcores=NUM_SUBCORES),
    compiler_params=pltpu.CompilerParams(use_tc_tiling_on_sc=False),
    scratch_types=[
        pltpu.VMEM((SENT_PER_W * L,), jnp.int32),
        pltpu.VMEM((NBUF, L, 2 * EMBED), jnp.float32),
        pltpu.VMEM((SENT_PER_W, EMBED), jnp.float32),
        pltpu.SemaphoreType.DMA,
        pltpu.SemaphoreType.DMA,
        pltpu.SemaphoreType.DMA,
        pltpu.SemaphoreType.DMA,
    ],
)(_gather_sums_body)


def _mlp_body(x_ref, sums_ref, w1_ref, b1_ref, w2_ref, b2_ref, out_ref):
    cnt = jnp.sum((x_ref[...] != 0).astype(jnp.float32), axis=1, keepdims=True)
    pooled = sums_ref[...] / (cnt + 1e-8)
    h = jnp.maximum(
        jnp.dot(pooled, w1_ref[...], preferred_element_type=jnp.float32)
        + b1_ref[...], 0.0)
    out_ref[...] = (
        jnp.dot(h, w2_ref[...], preferred_element_type=jnp.float32)
        + b2_ref[...])


BLK = 512


def kernel(x, table, W1, b1, W2, b2):
    pairs = pl.pallas_call(
        _transpose_body,
        grid=((VOCAB + TCC - 1) // TCC,),
        in_specs=[pl.BlockSpec((EMBED, TCC), lambda i: (0, i))],
        out_specs=pl.BlockSpec((TCC, 2 * EMBED), lambda i: (i, 0)),
        out_shape=jax.ShapeDtypeStruct((VOCAB, 2 * EMBED), jnp.float32),
    )(table.T)
    sums = _gather_sums(pairs, x.reshape(-1))
    out = pl.pallas_call(
        _mlp_body,
        grid=(B // BLK,),
        in_specs=[
            pl.BlockSpec((BLK, L), lambda i: (i, 0)),
            pl.BlockSpec((BLK, EMBED), lambda i: (i, 0)),
            pl.BlockSpec((EMBED, HIDDEN), lambda i: (0, 0)),
            pl.BlockSpec((1, HIDDEN), lambda i: (0, 0)),
            pl.BlockSpec((HIDDEN, EMBED), lambda i: (0, 0)),
            pl.BlockSpec((1, EMBED), lambda i: (0, 0)),
        ],
        out_specs=pl.BlockSpec((BLK, EMBED), lambda i: (i, 0)),
        out_shape=jax.ShapeDtypeStruct((B, EMBED), jnp.float32),
    )(x, sums, W1, b1.reshape(1, HIDDEN), W2, b2.reshape(1, EMBED))
    return out


# speedup vs baseline: 1.1998x; 1.0767x over previous
import functools

import jax
import jax.numpy as jnp
from jax import lax
from jax.experimental import pallas as pl
from jax.experimental.pallas import tpu as pltpu
from jax.experimental.pallas import tpu_sc as plsc

B = 4096
L = 200
EMBED = 64
HIDDEN = 128

NUM_CORES = 2
NUM_SUBCORES = 16
NW = NUM_CORES * NUM_SUBCORES
SENT_PER_W = B // NW
CHUNK0 = 128
CHUNK1 = L - CHUNK0


NBUF = 2
RUNROLL = 8
VOCAB = 1000000
TCC = 8192


def _transpose_body(tt_ref, out_ref):
    t = tt_ref[...]
    ez = jnp.concatenate(
        [jnp.eye(EMBED, dtype=jnp.float32),
         jnp.zeros((EMBED, EMBED), jnp.float32)], axis=1)
    out_ref[...] = lax.dot_general(
        t, ez, (((0,), (0,)), ((), ())),
        precision=lax.Precision.HIGHEST,
        preferred_element_type=jnp.float32)


def _gather_sums_body(pairs_hbm, x_hbm, sums_hbm, idx_v, rows_v,
                      acc_v, sem0, sem1, sem2, sem3):
    sems = (sem0, sem1, sem2, sem3)
    wid = lax.axis_index("s") * NUM_CORES + lax.axis_index("c")
    base = wid * SENT_PER_W
    pltpu.sync_copy(x_hbm.at[pl.ds(base * L, SENT_PER_W * L)], idx_v)

    def fire(s, b):
        off = pl.multiple_of(s * L, 8)
        pltpu.async_copy(
            pairs_hbm.at[idx_v.at[pl.ds(off, CHUNK0)]],
            rows_v.at[b, pl.ds(0, CHUNK0)], sems[b])
        pltpu.async_copy(
            pairs_hbm.at[idx_v.at[pl.ds(off + CHUNK0, CHUNK1)]],
            rows_v.at[b, pl.ds(CHUNK0, CHUNK1)], sems[b])

    def wait(b):
        pltpu.make_async_copy(
            pairs_hbm.at[idx_v.at[pl.ds(0, CHUNK0)]],
            rows_v.at[b, pl.ds(0, CHUNK0)], sems[b]).wait()
        pltpu.make_async_copy(
            pairs_hbm.at[idx_v.at[pl.ds(0, CHUNK1)]],
            rows_v.at[b, pl.ds(CHUNK0, CHUNK1)], sems[b]).wait()

    def reduce(s, b):
        def red(t, accs):
            r0 = t * RUNROLL
            for dr in range(RUNROLL):
                accs = tuple(accs[c] + rows_v[b, r0 + dr, pl.ds(c * 16, 16)]
                             for c in range(EMBED // 16))
            return accs

        z = jnp.zeros((16,), jnp.float32)
        accs = lax.fori_loop(0, L // RUNROLL, red, (z, z, z, z))
        for c in range(EMBED // 16):
            acc_v[s, pl.ds(c * 16, 16)] = accs[c]

    for b in range(NBUF):
        fire(b, b)

    def group(g, carry):
        s0 = g * NBUF
        for b in range(NBUF):
            wait(b)
            reduce(s0 + b, b)
            fire(s0 + b + NBUF, b)
        return carry

    lax.fori_loop(0, SENT_PER_W // NBUF - 1, group, 0)
    s0 = SENT_PER_W - NBUF
    for b in range(NBUF):
        wait(b)
        reduce(s0 + b, b)

    pltpu.sync_copy(acc_v, sums_hbm.at[pl.ds(base, SENT_PER_W)])


_gather_sums = functools.partial(
    pl.kernel,
    out_type=jax.ShapeDtypeStruct((B, EMBED), jnp.float32),
    mesh=plsc.VectorSubcoreMesh(core_axis_name="c", subcore_axis_name="s",
                                num_cores=NUM_CORES,
                                num_subcores=NUM_SUBCORES),
    compiler_params=pltpu.CompilerParams(use_tc_tiling_on_sc=False),
    scratch_types=[
        pltpu.VMEM((SENT_PER_W * L,), jnp.int32),
        pltpu.VMEM((NBUF, L, 2 * EMBED), jnp.float32),
        pltpu.VMEM((SENT_PER_W, EMBED), jnp.float32),
        pltpu.SemaphoreType.DMA,
        pltpu.SemaphoreType.DMA,
        pltpu.SemaphoreType.DMA,
        pltpu.SemaphoreType.DMA,
    ],
)(_gather_sums_body)


def _mlp_body(x_ref, sums_ref, w1_ref, b1_ref, w2_ref, b2_ref, out_ref):
    cnt = jnp.sum((x_ref[...] != 0).astype(jnp.float32), axis=1, keepdims=True)
    pooled = sums_ref[...] / (cnt + 1e-8)
    h = jnp.maximum(
        jnp.dot(pooled, w1_ref[...], preferred_element_type=jnp.float32)
        + b1_ref[...], 0.0)
    out_ref[...] = (
        jnp.dot(h, w2_ref[...], preferred_element_type=jnp.float32)
        + b2_ref[...])


BLK = 512


def kernel(x, table, W1, b1, W2, b2):
    pairs = pl.pallas_call(
        _transpose_body,
        grid=((VOCAB + TCC - 1) // TCC,),
        in_specs=[pl.BlockSpec((EMBED, TCC), lambda i: (0, i))],
        out_specs=pl.BlockSpec((TCC, 2 * EMBED), lambda i: (i, 0)),
        out_shape=jax.ShapeDtypeStruct((VOCAB, 2 * EMBED), jnp.float32),
    )(table.T)
    sums = _gather_sums(pairs, x.reshape(-1))
    out = pl.pallas_call(
        _mlp_body,
        grid=(B // BLK,),
        in_specs=[
            pl.BlockSpec((BLK, L), lambda i: (i, 0)),
            pl.BlockSpec((BLK, EMBED), lambda i: (i, 0)),
            pl.BlockSpec((EMBED, HIDDEN), lambda i: (0, 0)),
            pl.BlockSpec((1, HIDDEN), lambda i: (0, 0)),
            pl.BlockSpec((HIDDEN, EMBED), lambda i: (0, 0)),
            pl.BlockSpec((1, EMBED), lambda i: (0, 0)),
        ],
        out_specs=pl.BlockSpec((BLK, EMBED), lambda i: (i, 0)),
        out_shape=jax.ShapeDtypeStruct((B, EMBED), jnp.float32),
    )(x, sums, W1, b1.reshape(1, HIDDEN), W2, b2.reshape(1, EMBED))
    return out


# speedup vs baseline: 2.2661x; 1.8887x over previous
import functools

import jax
import jax.numpy as jnp
from jax import lax
from jax.experimental import pallas as pl
from jax.experimental.pallas import tpu as pltpu
from jax.experimental.pallas import tpu_sc as plsc

B = 4096
L = 200
EMBED = 64
HIDDEN = 128

NUM_CORES = 2
NUM_SUBCORES = 16
NW = NUM_CORES * NUM_SUBCORES
SENT_PER_W = B // NW
CHUNK0 = 128
CHUNK1 = L - CHUNK0


NBUF = 4
RUNROLL = 8
VOCAB = 1000000
TCC = 8192


H0 = 524288


def _transpose_body(ta_ref, tb_ref, out_ref):
    eye = jnp.eye(EMBED, dtype=jnp.bfloat16)

    def dot(a):
        return lax.dot_general(a, eye, (((0,), (0,)), ((), ())),
                               preferred_element_type=jnp.float32)

    def tr(t):
        hi = t.astype(jnp.bfloat16)
        lo = (t - hi.astype(jnp.float32)).astype(jnp.bfloat16)
        return dot(hi) + dot(lo)

    out_ref[...] = jnp.concatenate([tr(ta_ref[...]), tr(tb_ref[...])],
                                   axis=1)


def _gather_sums_body(pairs_hbm, x_hbm, sums_hbm, idx_v, rows_v,
                      acc_v, sem0, sem1, sem2, sem3):
    sems = (sem0, sem1, sem2, sem3)
    wid = lax.axis_index("s") * NUM_CORES + lax.axis_index("c")
    base = wid * SENT_PER_W
    pltpu.sync_copy(x_hbm.at[pl.ds(base * L, SENT_PER_W * L)], idx_v)

    def remap(j, carry):
        v = idx_v[pl.ds(j * 16, 16)]
        idx_v[pl.ds(j * 16, 16)] = ((v & (H0 - 1)) << 1) | (v >> 19)
        return carry

    lax.fori_loop(0, SENT_PER_W * L // 16, remap, 0)

    def fire(s, b):
        off = pl.multiple_of(s * L, 8)
        pltpu.async_copy(
            pairs_hbm.at[idx_v.at[pl.ds(off, CHUNK0)]],
            rows_v.at[b, pl.ds(0, CHUNK0)], sems[b])
        pltpu.async_copy(
            pairs_hbm.at[idx_v.at[pl.ds(off + CHUNK0, CHUNK1)]],
            rows_v.at[b, pl.ds(CHUNK0, CHUNK1)], sems[b])

    def wait(b):
        pltpu.make_async_copy(
            pairs_hbm.at[idx_v.at[pl.ds(0, CHUNK0)]],
            rows_v.at[b, pl.ds(0, CHUNK0)], sems[b]).wait()
        pltpu.make_async_copy(
            pairs_hbm.at[idx_v.at[pl.ds(0, CHUNK1)]],
            rows_v.at[b, pl.ds(CHUNK0, CHUNK1)], sems[b]).wait()

    def reduce(s, b):
        def red(t, accs):
            r0 = t * RUNROLL
            for dr in range(RUNROLL):
                accs = tuple(accs[c] + rows_v[b, r0 + dr, pl.ds(c * 16, 16)]
                             for c in range(EMBED // 16))
            return accs

        z = jnp.zeros((16,), jnp.float32)
        accs = lax.fori_loop(0, L // RUNROLL, red, (z, z, z, z))
        for c in range(EMBED // 16):
            acc_v[s, pl.ds(c * 16, 16)] = accs[c]

    for b in range(NBUF):
        fire(b, b)

    def group(g, carry):
        s0 = g * NBUF
        for b in range(NBUF):
            wait(b)
            reduce(s0 + b, b)
            fire(s0 + b + NBUF, b)
        return carry

    lax.fori_loop(0, SENT_PER_W // NBUF - 1, group, 0)
    s0 = SENT_PER_W - NBUF
    for b in range(NBUF):
        wait(b)
        reduce(s0 + b, b)

    pltpu.sync_copy(acc_v, sums_hbm.at[pl.ds(base, SENT_PER_W)])


_gather_sums = functools.partial(
    pl.kernel,
    out_type=jax.ShapeDtypeStruct((B, EMBED), jnp.float32),
    mesh=plsc.VectorSubcoreMesh(core_axis_name="c", subcore_axis_name="s",
                                num_cores=NUM_CORES,
                                num_subcores=NUM_SUBCORES),
    compiler_params=pltpu.CompilerParams(use_tc_tiling_on_sc=False),
    scratch_types=[
        pltpu.VMEM((SENT_PER_W * L,), jnp.int32),
        pltpu.VMEM((NBUF, L, EMBED), jnp.float32),
        pltpu.VMEM((SENT_PER_W, EMBED), jnp.float32),
        pltpu.SemaphoreType.DMA,
        pltpu.SemaphoreType.DMA,
        pltpu.SemaphoreType.DMA,
        pltpu.SemaphoreType.DMA,
    ],
)(_gather_sums_body)

_NTBLK = H0 // TCC


def _mlp_body(x_ref, sums_ref, w1_ref, b1_ref, w2_ref, b2_ref, out_ref):
    cnt = jnp.sum((x_ref[...] != 0).astype(jnp.float32), axis=1, keepdims=True)
    pooled = sums_ref[...] / (cnt + 1e-8)
    h = jnp.maximum(
        jnp.dot(pooled, w1_ref[...], preferred_element_type=jnp.float32)
        + b1_ref[...], 0.0)
    out_ref[...] = (
        jnp.dot(h, w2_ref[...], preferred_element_type=jnp.float32)
        + b2_ref[...])


BLK = 512


def kernel(x, table, W1, b1, W2, b2):
    pairs = pl.pallas_call(
        _transpose_body,
        grid=(_NTBLK,),
        in_specs=[
            pl.BlockSpec((EMBED, TCC), lambda i: (0, i)),
            pl.BlockSpec((EMBED, TCC),
                         lambda i: (0, jnp.minimum(i + _NTBLK,
                                                   (VOCAB - 1) // TCC))),
        ],
        out_specs=pl.BlockSpec((TCC, 2 * EMBED), lambda i: (i, 0)),
        out_shape=jax.ShapeDtypeStruct((H0, 2 * EMBED), jnp.float32),
    )(table.T, table.T)
    tbl2 = pairs.reshape(2 * H0, EMBED)
    sums = _gather_sums(tbl2, x.reshape(-1))
    out = pl.pallas_call(
        _mlp_body,
        grid=(B // BLK,),
        in_specs=[
            pl.BlockSpec((BLK, L), lambda i: (i, 0)),
            pl.BlockSpec((BLK, EMBED), lambda i: (i, 0)),
            pl.BlockSpec((EMBED, HIDDEN), lambda i: (0, 0)),
            pl.BlockSpec((1, HIDDEN), lambda i: (0, 0)),
            pl.BlockSpec((HIDDEN, EMBED), lambda i: (0, 0)),
            pl.BlockSpec((1, EMBED), lambda i: (0, 0)),
        ],
        out_specs=pl.BlockSpec((BLK, EMBED), lambda i: (i, 0)),
        out_shape=jax.ShapeDtypeStruct((B, EMBED), jnp.float32),
    )(x, sums, W1, b1.reshape(1, HIDDEN), W2, b2.reshape(1, EMBED))
    return out


# speedup vs baseline: 2.3576x; 1.0404x over previous
import functools

import jax
import jax.numpy as jnp
from jax import lax
from jax.experimental import pallas as pl
from jax.experimental.pallas import tpu as pltpu
from jax.experimental.pallas import tpu_sc as plsc

B = 4096
L = 200
EMBED = 64
HIDDEN = 128

NUM_CORES = 2
NUM_SUBCORES = 16
NW = NUM_CORES * NUM_SUBCORES
SENT_PER_W = B // NW
CHUNK0 = 128
CHUNK1 = L - CHUNK0


NBUF = 4
RUNROLL = 8
VOCAB = 1000000
TCC = 16384


H0 = 524288


def _transpose_body(ta_ref, tb_ref, out_ref):
    eye = jnp.eye(EMBED, dtype=jnp.bfloat16)

    def dot(a):
        return lax.dot_general(a, eye, (((0,), (0,)), ((), ())),
                               preferred_element_type=jnp.float32)

    def tr(t):
        hi = t.astype(jnp.bfloat16)
        lo = (t - hi.astype(jnp.float32)).astype(jnp.bfloat16)
        return dot(hi) + dot(lo)

    out_ref[...] = jnp.concatenate([tr(ta_ref[...]), tr(tb_ref[...])],
                                   axis=1)


def _gather_sums_body(pairs_hbm, x_hbm, sums_hbm, idx_v, rows_v,
                      acc_v, sem0, sem1, sem2, sem3):
    sems = (sem0, sem1, sem2, sem3)
    wid = lax.axis_index("s") * NUM_CORES + lax.axis_index("c")
    base = wid * SENT_PER_W
    pltpu.sync_copy(x_hbm.at[pl.ds(base * L, SENT_PER_W * L)], idx_v)

    def remap(j, carry):
        v = idx_v[pl.ds(j * 16, 16)]
        idx_v[pl.ds(j * 16, 16)] = ((v & (H0 - 1)) << 1) | (v >> 19)
        return carry

    lax.fori_loop(0, SENT_PER_W * L // 16, remap, 0)

    def fire(s, b):
        off = pl.multiple_of(s * L, 8)
        pltpu.async_copy(
            pairs_hbm.at[idx_v.at[pl.ds(off, CHUNK0)]],
            rows_v.at[b, pl.ds(0, CHUNK0)], sems[b])
        pltpu.async_copy(
            pairs_hbm.at[idx_v.at[pl.ds(off + CHUNK0, CHUNK1)]],
            rows_v.at[b, pl.ds(CHUNK0, CHUNK1)], sems[b])

    def wait(b):
        pltpu.make_async_copy(
            pairs_hbm.at[idx_v.at[pl.ds(0, CHUNK0)]],
            rows_v.at[b, pl.ds(0, CHUNK0)], sems[b]).wait()
        pltpu.make_async_copy(
            pairs_hbm.at[idx_v.at[pl.ds(0, CHUNK1)]],
            rows_v.at[b, pl.ds(CHUNK0, CHUNK1)], sems[b]).wait()

    def reduce(s, b):
        def red(t, accs):
            r0 = t * RUNROLL
            for dr in range(RUNROLL):
                accs = tuple(accs[c] + rows_v[b, r0 + dr, pl.ds(c * 16, 16)]
                             for c in range(EMBED // 16))
            return accs

        z = jnp.zeros((16,), jnp.float32)
        accs = lax.fori_loop(0, L // RUNROLL, red, (z, z, z, z))
        for c in range(EMBED // 16):
            acc_v[s, pl.ds(c * 16, 16)] = accs[c]

    for b in range(NBUF):
        fire(b, b)

    def group(g, carry):
        s0 = g * NBUF
        for b in range(NBUF):
            wait(b)
            reduce(s0 + b, b)
            fire(s0 + b + NBUF, b)
        return carry

    lax.fori_loop(0, SENT_PER_W // NBUF - 1, group, 0)
    s0 = SENT_PER_W - NBUF
    for b in range(NBUF):
        wait(b)
        reduce(s0 + b, b)

    pltpu.sync_copy(acc_v, sums_hbm.at[pl.ds(base, SENT_PER_W)])


_gather_sums = functools.partial(
    pl.kernel,
    out_type=jax.ShapeDtypeStruct((B, EMBED), jnp.float32),
    mesh=plsc.VectorSubcoreMesh(core_axis_name="c", subcore_axis_name="s",
                                num_cores=NUM_CORES,
                                num_subcores=NUM_SUBCORES),
    compiler_params=pltpu.CompilerParams(use_tc_tiling_on_sc=False),
    scratch_types=[
        pltpu.VMEM((SENT_PER_W * L,), jnp.int32),
        pltpu.VMEM((NBUF, L, EMBED), jnp.float32),
        pltpu.VMEM((SENT_PER_W, EMBED), jnp.float32),
        pltpu.SemaphoreType.DMA,
        pltpu.SemaphoreType.DMA,
        pltpu.SemaphoreType.DMA,
        pltpu.SemaphoreType.DMA,
    ],
)(_gather_sums_body)

_NTBLK = H0 // TCC


def _mlp_body(x_ref, sums_ref, w1_ref, b1_ref, w2_ref, b2_ref, out_ref):
    cnt = jnp.sum((x_ref[...] != 0).astype(jnp.float32), axis=1, keepdims=True)
    pooled = sums_ref[...] / (cnt + 1e-8)
    h = jnp.maximum(
        jnp.dot(pooled, w1_ref[...], preferred_element_type=jnp.float32)
        + b1_ref[...], 0.0)
    out_ref[...] = (
        jnp.dot(h, w2_ref[...], preferred_element_type=jnp.float32)
        + b2_ref[...])


BLK = 512


def kernel(x, table, W1, b1, W2, b2):
    pairs = pl.pallas_call(
        _transpose_body,
        grid=(_NTBLK,),
        in_specs=[
            pl.BlockSpec((EMBED, TCC), lambda i: (0, i)),
            pl.BlockSpec((EMBED, TCC),
                         lambda i: (0, jnp.minimum(i + _NTBLK,
                                                   (VOCAB - 1) // TCC))),
        ],
        out_specs=pl.BlockSpec((TCC, 2 * EMBED), lambda i: (i, 0)),
        out_shape=jax.ShapeDtypeStruct((H0, 2 * EMBED), jnp.float32),
    )(table.T, table.T)
    tbl2 = pairs.reshape(2 * H0, EMBED)
    sums = _gather_sums(tbl2, x.reshape(-1))
    out = pl.pallas_call(
        _mlp_body,
        grid=(B // BLK,),
        in_specs=[
            pl.BlockSpec((BLK, L), lambda i: (i, 0)),
            pl.BlockSpec((BLK, EMBED), lambda i: (i, 0)),
            pl.BlockSpec((EMBED, HIDDEN), lambda i: (0, 0)),
            pl.BlockSpec((1, HIDDEN), lambda i: (0, 0)),
            pl.BlockSpec((HIDDEN, EMBED), lambda i: (0, 0)),
            pl.BlockSpec((1, EMBED), lambda i: (0, 0)),
        ],
        out_specs=pl.BlockSpec((BLK, EMBED), lambda i: (i, 0)),
        out_shape=jax.ShapeDtypeStruct((B, EMBED), jnp.float32),
    )(x, sums, W1, b1.reshape(1, HIDDEN), W2, b2.reshape(1, EMBED))
    return out
